# baseline (speedup 1.0000x reference)
import functools
import math

import jax
import jax.numpy as jnp
from jax.experimental import pallas as pl
from jax.experimental.pallas import tpu as pltpu

_INV_PI = 0.31830987334251404
_MAGIC = 12582912.0
_PI1 = 3.140625
_PI2 = 0.0009676535846665502
_S3 = -0.1666584
_S5 = 0.00831458
_S7 = -0.00018561
_C2 = -0.49994812
_C4 = 0.04152708
_C6 = -0.00128256


def _ffT_kernel(x_ref, wt_ref, o_ref, *, enc_dim):
    f32 = jnp.float32
    wt = wt_ref[...]
    for t in range(x_ref.shape[0]):
        proj = jnp.dot(wt, x_ref[t], preferred_element_type=jnp.float32)
        tt = proj * f32(_INV_PI) + f32(_MAGIC)
        sign = jax.lax.shift_left(
            jax.lax.bitwise_and(jax.lax.bitcast_convert_type(tt, jnp.int32), 1), 31)
        kf = tt - f32(_MAGIC)
        r = proj - kf * f32(_PI1)
        r = r - kf * f32(_PI2)
        r2 = r * r
        sinp = r * (f32(1.0) + r2 * (f32(_S3) + r2 * (f32(_S5) + r2 * f32(_S7))))
        cosp = f32(1.0) + r2 * (f32(_C2) + r2 * (f32(_C4) + r2 * f32(_C6)))
        o_ref[t, :enc_dim] = jax.lax.bitcast_convert_type(
            jax.lax.bitwise_xor(jax.lax.bitcast_convert_type(sinp, jnp.int32), sign),
            jnp.float32).astype(o_ref.dtype)
        o_ref[t, enc_dim:] = jax.lax.bitcast_convert_type(
            jax.lax.bitwise_xor(jax.lax.bitcast_convert_type(cosp, jnp.int32), sign),
            jnp.float32).astype(o_ref.dtype)


def kernel(pos, B):
    pos_dim, enc_dim = B.shape
    f_dim = 2 * enc_dim
    assert pos.shape[-1] == pos_dim and pos.ndim == 3
    n, s = pos.shape[0], pos.shape[1]
    out_dtype = pos.dtype

    x_t = jnp.transpose(pos, (0, 2, 1))
    wt = B.T.astype(pos.dtype)

    tb = 32
    while n % tb:
        tb //= 2
    n_tiles = n // tb

    out_t = pl.pallas_call(
        functools.partial(_ffT_kernel, enc_dim=enc_dim),
        out_shape=jax.ShapeDtypeStruct((n, f_dim, s), out_dtype),
        grid=(n_tiles,),
        in_specs=[
            pl.BlockSpec((tb, pos_dim, s), lambda i: (i, 0, 0)),
            pl.BlockSpec((enc_dim, pos_dim), lambda i: (0, 0)),
        ],
        out_specs=pl.BlockSpec((tb, f_dim, s), lambda i: (i, 0, 0)),
        compiler_params=pltpu.CompilerParams(
            dimension_semantics=("parallel",),
            vmem_limit_bytes=100 * 1024 * 1024,
        ),
    )(x_t, wt)
    return jnp.transpose(out_t, (0, 2, 1))


# speedup vs baseline: 1.4093x; 1.4093x over previous
import functools
import math

import jax
import jax.numpy as jnp
from jax.experimental import pallas as pl
from jax.experimental.pallas import tpu as pltpu

_INV_PI = 0.31830987334251404
_MAGIC = 12582912.0
_PI1 = 3.140625
_PI2 = 0.0009676535846665502
_S3 = -0.1666584
_S5 = 0.00831458
_S7 = -0.00018561
_C2 = -0.49994812
_C4 = 0.04152708
_C6 = -0.00128256


def _ffT_kernel(x_ref, wt_ref, o_ref, *, enc_dim):
    f32 = jnp.float32
    wt = wt_ref[...]
    for t in range(x_ref.shape[0]):
        proj = jnp.dot(wt, x_ref[t], preferred_element_type=jnp.float32)
        tt = proj * f32(_INV_PI) + f32(_MAGIC)
        sign = jax.lax.shift_left(
            jax.lax.bitwise_and(jax.lax.bitcast_convert_type(tt, jnp.int32), 1), 31)
        kf = tt - f32(_MAGIC)
        r = proj - kf * f32(_PI1)
        r = r - kf * f32(_PI2)
        r2 = r * r
        sinp = r * (f32(1.0) + r2 * (f32(_S3) + r2 * (f32(_S5) + r2 * f32(_S7))))
        cosp = f32(1.0) + r2 * (f32(_C2) + r2 * (f32(_C4) + r2 * f32(_C6)))
        o_ref[t, :enc_dim] = jax.lax.bitcast_convert_type(
            jax.lax.bitwise_xor(jax.lax.bitcast_convert_type(sinp, jnp.int32), sign),
            jnp.float32).astype(o_ref.dtype)
        o_ref[t, enc_dim:] = jax.lax.bitcast_convert_type(
            jax.lax.bitwise_xor(jax.lax.bitcast_convert_type(cosp, jnp.int32), sign),
            jnp.float32).astype(o_ref.dtype)


def kernel(pos, B):
    pos_dim, enc_dim = B.shape
    f_dim = 2 * enc_dim
    assert pos.shape[-1] == pos_dim and pos.ndim == 3
    n, s = pos.shape[0], pos.shape[1]
    out_dtype = pos.dtype

    x_t = jnp.transpose(pos, (0, 2, 1))
    wt = B.T.astype(pos.dtype)

    tb = 128
    while n % tb:
        tb //= 2
    n_tiles = n // tb

    out_t = pl.pallas_call(
        functools.partial(_ffT_kernel, enc_dim=enc_dim),
        out_shape=jax.ShapeDtypeStruct((n, f_dim, s), out_dtype),
        grid=(n_tiles,),
        in_specs=[
            pl.BlockSpec((tb, pos_dim, s), lambda i: (i, 0, 0)),
            pl.BlockSpec((enc_dim, pos_dim), lambda i: (0, 0)),
        ],
        out_specs=pl.BlockSpec((tb, f_dim, s), lambda i: (i, 0, 0)),
        compiler_params=pltpu.CompilerParams(
            dimension_semantics=("parallel",),
            vmem_limit_bytes=100 * 1024 * 1024,
        ),
    )(x_t, wt)
    return jnp.transpose(out_t, (0, 2, 1))
